# baseline (speedup 1.0000x reference)
import functools

import jax
import jax.numpy as jnp
from jax import lax
from jax.experimental import pallas as pl
from jax.experimental.pallas import tpu as pltpu
from jax.experimental.pallas import tpu_sc as plsc

N = 10000
E = 320000
P = 100000
IN_CH = 128
HIDDEN = 256

NC = 2
NS = 16
NPAD = 10240
RPT = NPAD // NS
F = 64

K_SEG = 80
CPT = 252
EPT = K_SEG * CPT
EPAD = EPT * NS
SJ = 36
SB = CPT // SJ

K_DEC = 96
DCH = 36
PPT = K_DEC * DCH
PPAD = PPT * NC * NS


def _fill(ref, val, rows, cols):
    v = jnp.full((16,), val, jnp.float32)

    def row(r, carry):
        def col(k, carry2):
            ref[r, pl.ds(k * 16, 16)] = v
            return carry2
        return lax.fori_loop(0, cols // 16, col, carry)

    lax.fori_loop(0, rows, row, 0)


def _fill_1d(ref, val, n):
    v = jnp.full((16,), val, jnp.float32)

    def it(k, carry):
        ref[pl.ds(k * 16, 16)] = v
        return carry

    lax.fori_loop(0, n // 16, it, 0)


def _make_segsum(nq, with_cnt):
    qpc = nq // NC
    mesh = plsc.VectorSubcoreMesh(core_axis_name="c", subcore_axis_name="s")

    out_type = [jax.ShapeDtypeStruct((nq, NPAD, F), jnp.float32)]
    if with_cnt:
        out_type.append(jax.ShapeDtypeStruct((NPAD,), jnp.float32))

    scratch = [
        pltpu.VMEM_SHARED((NPAD, F), jnp.float32),
        pltpu.VMEM_SHARED((NPAD, F), jnp.float32),
        pltpu.VMEM((SJ, K_SEG), jnp.int32),
        pltpu.VMEM((SJ, K_SEG), jnp.int32),
    ] + [pltpu.VMEM((K_SEG, F), jnp.float32) for _ in range(4)] \
      + [pltpu.SemaphoreType.DMA for _ in range(8)]
    if with_cnt:
        scratch += [
            pltpu.VMEM_SHARED((NPAD,), jnp.float32),
            pltpu.VMEM((K_SEG,), jnp.float32),
        ] + [pltpu.SemaphoreType.DMA for _ in range(4)]

    def body(tab_hbm, src_hbm, dst_hbm, out_hbm, *rest):
        if with_cnt:
            (cnt_hbm, tab_s, acc_s, sidx_blk, didx_blk,
             rv0, rv1, rv2, rv3,
             sg0, sg1, sg2, sg3, ss0, ss1, ss2, ss3,
             cnt_s, ones_v, sc0, sc1, sc2, sc3) = rest
        else:
            (tab_s, acc_s, sidx_blk, didx_blk,
             rv0, rv1, rv2, rv3,
             sg0, sg1, sg2, sg3, ss0, ss1, ss2, ss3) = rest
        c = lax.axis_index("c")
        s = lax.axis_index("s")
        r0 = s * RPT
        rvs = (rv0, rv1, rv2, rv3)
        sgs = (sg0, sg1, sg2, sg3)
        sss = (ss0, ss1, ss2, ss3)

        for qi in range(qpc):
            q = c * qpc + qi
            _fill(rv0, 0.0, K_SEG, F)
            pltpu.sync_copy(tab_hbm.at[q, pl.ds(r0, RPT)],
                            tab_s.at[pl.ds(r0, RPT)])
            for zb in range(RPT // K_SEG):
                pltpu.sync_copy(rv0,
                                acc_s.at[pl.ds(r0 + zb * K_SEG, K_SEG)])
            if with_cnt and qi == 0:
                _fill_1d(ones_v, 0.0, K_SEG)

                @pl.when(c == 0)
                def _():
                    for zb in range(RPT // K_SEG):
                        pltpu.sync_copy(
                            ones_v, cnt_s.at[pl.ds(r0 + zb * K_SEG, K_SEG)])
                _fill_1d(ones_v, 1.0, K_SEG)
            plsc.subcore_barrier()

            def sblk(u, carry):
                row0 = s * CPT + u * SJ
                pltpu.sync_copy(src_hbm.at[pl.ds(row0, SJ)], sidx_blk)
                pltpu.sync_copy(dst_hbm.at[pl.ds(row0, SJ)], didx_blk)

                def quad(i, carry2):
                    gs = []
                    for b in range(4):
                        gs.append(pltpu.async_copy(
                            tab_s.at[sidx_blk.at[4 * i + b]], rvs[b],
                            sgs[b]))
                    scs = []
                    for b in range(4):
                        gs[b].wait()
                        scs.append(pltpu.async_copy(
                            rvs[b], acc_s.at[didx_blk.at[4 * i + b]],
                            sss[b], add=True))
                    if with_cnt and qi == 0:
                        @pl.when(c == 0)
                        def _():
                            ccs = [pltpu.async_copy(
                                ones_v, cnt_s.at[didx_blk.at[4 * i + b]],
                                (sc0, sc1, sc2, sc3)[b], add=True)
                                for b in range(4)]
                            for cc in ccs:
                                cc.wait()
                    for sp in scs:
                        sp.wait()
                    return carry2

                lax.fori_loop(0, SJ // 4, quad, 0)
                return carry

            lax.fori_loop(0, SB, sblk, 0)
            plsc.subcore_barrier()

            pltpu.sync_copy(acc_s.at[pl.ds(r0, RPT)],
                            out_hbm.at[q, pl.ds(r0, RPT)])
            if with_cnt and qi == 0:
                @pl.when(c == 0)
                def _():
                    pltpu.sync_copy(cnt_s.at[pl.ds(r0, RPT)],
                                    cnt_hbm.at[pl.ds(r0, RPT)])

    return pl.kernel(body, out_type=tuple(out_type), mesh=mesh,
                     scratch_types=scratch,
                     compiler_params=pltpu.CompilerParams(
                         use_tc_tiling_on_sc=False))


_segsum2 = _make_segsum(2, True)
_segsum4 = _make_segsum(4, False)


RB = 512
NB = NPAD // RB


def _combine1_body(agg_ref, cnt_ref, x_ref, wl_ref, b_ref, wr_ref, out_ref):
    cnt = jnp.maximum(cnt_ref[...], 1.0)
    mean = jnp.concatenate([agg_ref[0], agg_ref[1]], axis=-1) / cnt
    h = (jnp.dot(mean, wl_ref[...], preferred_element_type=jnp.float32)
         + b_ref[...]
         + jnp.dot(x_ref[...], wr_ref[...],
                   preferred_element_type=jnp.float32))
    h = jnp.maximum(h, 0.0)
    for q in range(4):
        out_ref[q] = h[:, q * F:(q + 1) * F]


def _combine1(agg1, cnt2d, x_pad, W1_l, b1, W1_r):
    return pl.pallas_call(
        _combine1_body,
        grid=(NB,),
        in_specs=[
            pl.BlockSpec((2, RB, F), lambda i: (0, i, 0)),
            pl.BlockSpec((RB, 1), lambda i: (i, 0)),
            pl.BlockSpec((RB, IN_CH), lambda i: (i, 0)),
            pl.BlockSpec((IN_CH, HIDDEN), lambda i: (0, 0)),
            pl.BlockSpec((1, HIDDEN), lambda i: (0, 0)),
            pl.BlockSpec((IN_CH, HIDDEN), lambda i: (0, 0)),
        ],
        out_specs=pl.BlockSpec((4, RB, F), lambda i: (0, i, 0)),
        out_shape=jax.ShapeDtypeStruct((4, NPAD, F), jnp.float32),
    )(agg1, cnt2d, x_pad, W1_l, b1.reshape(1, HIDDEN), W1_r)


def _combine2_body(agg_ref, cnt_ref, h_ref, wl_ref, b_ref, wr_ref, out_ref):
    cnt = jnp.maximum(cnt_ref[...], 1.0)
    mean = jnp.concatenate([agg_ref[q] for q in range(4)], axis=-1) / cnt
    h = jnp.concatenate([h_ref[q] for q in range(4)], axis=-1)
    out_ref[...] = (
        jnp.dot(mean, wl_ref[...], preferred_element_type=jnp.float32)
        + b_ref[...]
        + jnp.dot(h, wr_ref[...], preferred_element_type=jnp.float32))


def _combine2(agg2, cnt2d, hT2, W2_l, b2, W2_r):
    return pl.pallas_call(
        _combine2_body,
        grid=(NB,),
        in_specs=[
            pl.BlockSpec((4, RB, F), lambda i: (0, i, 0)),
            pl.BlockSpec((RB, 1), lambda i: (i, 0)),
            pl.BlockSpec((4, RB, F), lambda i: (0, i, 0)),
            pl.BlockSpec((HIDDEN, HIDDEN), lambda i: (0, 0)),
            pl.BlockSpec((1, HIDDEN), lambda i: (0, 0)),
            pl.BlockSpec((HIDDEN, HIDDEN), lambda i: (0, 0)),
        ],
        out_specs=pl.BlockSpec((RB, HIDDEN), lambda i: (i, 0)),
        out_shape=jax.ShapeDtypeStruct((NPAD, HIDDEN), jnp.float32),
    )(agg2, cnt2d, hT2, W2_l, b2.reshape(1, HIDDEN), W2_r)


def _make_decode():
    mesh = plsc.VectorSubcoreMesh(core_axis_name="c", subcore_axis_name="s")
    scratch = (
        [pltpu.VMEM((DCH, K_DEC), jnp.int32) for _ in range(2)]
        + [pltpu.VMEM((K_DEC, HIDDEN // 2), jnp.int32) for _ in range(8)]
        + [pltpu.VMEM((PPT,), jnp.float32)]
        + [pltpu.SemaphoreType.DMA for _ in range(8)]
    )

    def body(z_hbm, es_hbm, ed_hbm, out_hbm, sidx_all, didx_all,
             zs0, zd0, zs1, zd1, zs2, zd2, zs3, zd3, outv,
             ga0, gb0, ga1, gb1, ga2, gb2, ga3, gb3):
        c = lax.axis_index("c")
        s = lax.axis_index("s")
        w = c * NS + s
        lanes = lax.iota(jnp.int32, 16)
        z16 = jnp.zeros((16,), jnp.float32)
        zss = (zs0, zs1, zs2, zs3)
        zds = (zd0, zd1, zd2, zd3)
        gas = (ga0, ga1, ga2, ga3)
        gbs = (gb0, gb1, gb2, gb3)

        pltpu.sync_copy(es_hbm.at[pl.ds(w * DCH, DCH)], sidx_all)
        pltpu.sync_copy(ed_hbm.at[pl.ds(w * DCH, DCH)], didx_all)

        def compute(zs, zd, j):
            def pair_loop(g, carry):
                vec = z16
                for i in range(16):
                    idx = g * 16 + i
                    himask = jnp.full((16,), -65536, jnp.int32)
                    terms = []
                    for t in range(HIDDEN // 32):
                        wa = zs[idx, pl.ds(t * 16, 16)]
                        wb = zd[idx, pl.ds(t * 16, 16)]
                        la = plsc.bitcast(wa << 16, jnp.float32)
                        lb = plsc.bitcast(wb << 16, jnp.float32)
                        ha = plsc.bitcast(wa & himask, jnp.float32)
                        hb = plsc.bitcast(wb & himask, jnp.float32)
                        terms.append(la * lb + ha * hb)
                    while len(terms) > 1:
                        terms = [terms[k] + terms[k + 1]
                                 for k in range(0, len(terms) - 1, 2)] + (
                                     [terms[-1]] if len(terms) % 2 else [])
                    vec = jnp.where(lanes == i, jnp.sum(terms[0]), vec)
                outv[pl.ds(j * K_DEC + g * 16, 16)] = vec
                return carry

            lax.fori_loop(0, K_DEC // 16, pair_loop, 0)

        def it(i, carry):
            cps = []
            for b in range(4):
                j = 4 * i + b
                cps.append((
                    pltpu.async_copy(z_hbm.at[sidx_all.at[j]], zss[b],
                                     gas[b]),
                    pltpu.async_copy(z_hbm.at[didx_all.at[j]], zds[b],
                                     gbs[b]),
                ))
            for b in range(4):
                cps[b][0].wait()
                cps[b][1].wait()
                compute(zss[b], zds[b], 4 * i + b)
            return carry

        lax.fori_loop(0, DCH // 4, it, 0)

        pltpu.sync_copy(outv, out_hbm.at[pl.ds(w * PPT, PPT)])

    return pl.kernel(body,
                     out_type=jax.ShapeDtypeStruct((PPAD,), jnp.float32),
                     mesh=mesh, scratch_types=scratch,
                     compiler_params=pltpu.CompilerParams(
                         use_tc_tiling_on_sc=False,
                         needs_layout_passes=False))


_decode = _make_decode()


def kernel(x, edge_index, edges, W1_l, b1, W1_r, W2_l, b2, W2_r):
    pad_e = EPAD - E
    pad_src = (jnp.arange(pad_e, dtype=jnp.int32) * 97) % N
    pad_dst = N + (jnp.arange(pad_e, dtype=jnp.int32) % (NPAD - N))
    src2 = jnp.concatenate([edge_index[0], pad_src]).reshape(
        EPAD // K_SEG, K_SEG)
    dst2 = jnp.concatenate([edge_index[1], pad_dst]).reshape(
        EPAD // K_SEG, K_SEG)
    x_pad = jnp.pad(x, ((0, NPAD - N), (0, 0)))
    xT2 = x_pad.reshape(NPAD, 2, F).transpose(1, 0, 2)

    agg1, cnt = _segsum2(xT2, src2, dst2)
    cnt2d = cnt.reshape(NPAD, 1)
    hT2 = _combine1(agg1, cnt2d, x_pad, W1_l, b1, W1_r)
    (agg2,) = _segsum4(hT2, src2, dst2)
    z = _combine2(agg2, cnt2d, hT2, W2_l, b2, W2_r)

    pad_p = PPAD - P
    pad_idx = (jnp.arange(pad_p, dtype=jnp.int32) * 89) % N
    es2 = jnp.concatenate([edges[:, 0], pad_idx]).reshape(
        PPAD // K_DEC, K_DEC)
    ed2 = jnp.concatenate([edges[:, 1], pad_idx]).reshape(
        PPAD // K_DEC, K_DEC)
    zi = lax.bitcast_convert_type(
        z.astype(jnp.bfloat16).reshape(NPAD, HIDDEN // 2, 2), jnp.int32)
    out = _decode(zi, es2, ed2)
    return out[:P]


# speedup vs baseline: 1.0239x; 1.0239x over previous
import functools

import jax
import jax.numpy as jnp
from jax import lax
from jax.experimental import pallas as pl
from jax.experimental.pallas import tpu as pltpu
from jax.experimental.pallas import tpu_sc as plsc

N = 10000
E = 320000
P = 100000
IN_CH = 128
HIDDEN = 256

NC = 2
NS = 16
NPAD = 10240
RPT = NPAD // NS
F = 64

K_SEG = 80
CPT = 252
EPT = K_SEG * CPT
EPAD = EPT * NS
SJ = 36
SB = CPT // SJ

K_DEC = 96
DCH = 36
PPT = K_DEC * DCH
PPAD = PPT * NC * NS


def _fill(ref, val, rows, cols):
    v = jnp.full((16,), val, jnp.float32)

    def row(r, carry):
        def col(k, carry2):
            ref[r, pl.ds(k * 16, 16)] = v
            return carry2
        return lax.fori_loop(0, cols // 16, col, carry)

    lax.fori_loop(0, rows, row, 0)


def _fill_1d(ref, val, n):
    v = jnp.full((16,), val, jnp.float32)

    def it(k, carry):
        ref[pl.ds(k * 16, 16)] = v
        return carry

    lax.fori_loop(0, n // 16, it, 0)


def _make_segsum(nq, with_cnt):
    qpc = nq // NC
    mesh = plsc.VectorSubcoreMesh(core_axis_name="c", subcore_axis_name="s")

    out_type = [jax.ShapeDtypeStruct((nq, NPAD, F), jnp.float32)]
    if with_cnt:
        out_type.append(jax.ShapeDtypeStruct((NPAD,), jnp.float32))

    scratch = [
        pltpu.VMEM_SHARED((NPAD, F), jnp.float32),
        pltpu.VMEM_SHARED((NPAD, F), jnp.float32),
        pltpu.VMEM((SJ, K_SEG), jnp.int32),
        pltpu.VMEM((SJ, K_SEG), jnp.int32),
    ] + [pltpu.VMEM((K_SEG, F), jnp.float32) for _ in range(4)] \
      + [pltpu.SemaphoreType.DMA for _ in range(8)]
    if with_cnt:
        scratch += [
            pltpu.VMEM_SHARED((NPAD,), jnp.float32),
            pltpu.VMEM((K_SEG,), jnp.float32),
        ] + [pltpu.SemaphoreType.DMA for _ in range(4)]

    def body(tab_hbm, src_hbm, dst_hbm, out_hbm, *rest):
        if with_cnt:
            (cnt_hbm, tab_s, acc_s, sidx_blk, didx_blk,
             rv0, rv1, rv2, rv3,
             sg0, sg1, sg2, sg3, ss0, ss1, ss2, ss3,
             cnt_s, ones_v, sc0, sc1, sc2, sc3) = rest
        else:
            (tab_s, acc_s, sidx_blk, didx_blk,
             rv0, rv1, rv2, rv3,
             sg0, sg1, sg2, sg3, ss0, ss1, ss2, ss3) = rest
        c = lax.axis_index("c")
        s = lax.axis_index("s")
        r0 = s * RPT
        rvs = (rv0, rv1, rv2, rv3)
        sgs = (sg0, sg1, sg2, sg3)
        sss = (ss0, ss1, ss2, ss3)

        for qi in range(qpc):
            q = c * qpc + qi
            _fill(rv0, 0.0, K_SEG, F)
            pltpu.sync_copy(tab_hbm.at[q, pl.ds(r0, RPT)],
                            tab_s.at[pl.ds(r0, RPT)])
            for zb in range(RPT // K_SEG):
                pltpu.sync_copy(rv0,
                                acc_s.at[pl.ds(r0 + zb * K_SEG, K_SEG)])
            if with_cnt and qi == 0:
                _fill_1d(ones_v, 0.0, K_SEG)

                @pl.when(c == 0)
                def _():
                    for zb in range(RPT // K_SEG):
                        pltpu.sync_copy(
                            ones_v, cnt_s.at[pl.ds(r0 + zb * K_SEG, K_SEG)])
                _fill_1d(ones_v, 1.0, K_SEG)
            plsc.subcore_barrier()

            def sblk(u, carry):
                row0 = s * CPT + u * SJ
                pltpu.sync_copy(src_hbm.at[pl.ds(row0, SJ)], sidx_blk)
                pltpu.sync_copy(dst_hbm.at[pl.ds(row0, SJ)], didx_blk)

                def quad(i, carry2):
                    gs = []
                    for b in range(4):
                        gs.append(pltpu.async_copy(
                            tab_s.at[sidx_blk.at[4 * i + b]], rvs[b],
                            sgs[b]))
                    scs = []
                    for b in range(4):
                        gs[b].wait()
                        scs.append(pltpu.async_copy(
                            rvs[b], acc_s.at[didx_blk.at[4 * i + b]],
                            sss[b], add=True))
                    if with_cnt and qi == 0:
                        @pl.when(c == 0)
                        def _():
                            ccs = [pltpu.async_copy(
                                ones_v, cnt_s.at[didx_blk.at[4 * i + b]],
                                (sc0, sc1, sc2, sc3)[b], add=True)
                                for b in range(4)]
                            for cc in ccs:
                                cc.wait()
                    for sp in scs:
                        sp.wait()
                    return carry2

                lax.fori_loop(0, SJ // 4, quad, 0)
                return carry

            lax.fori_loop(0, SB, sblk, 0)
            plsc.subcore_barrier()

            pltpu.sync_copy(acc_s.at[pl.ds(r0, RPT)],
                            out_hbm.at[q, pl.ds(r0, RPT)])
            if with_cnt and qi == 0:
                @pl.when(c == 0)
                def _():
                    pltpu.sync_copy(cnt_s.at[pl.ds(r0, RPT)],
                                    cnt_hbm.at[pl.ds(r0, RPT)])

    return pl.kernel(body, out_type=tuple(out_type), mesh=mesh,
                     scratch_types=scratch,
                     compiler_params=pltpu.CompilerParams(
                         use_tc_tiling_on_sc=False))


_segsum2 = _make_segsum(2, True)


FH = 128
SJ2 = 36


def _make_segsum_hbm():
    mesh = plsc.VectorSubcoreMesh(core_axis_name="c", subcore_axis_name="s")
    scratch = [
        pltpu.VMEM_SHARED((NPAD, FH), jnp.float32),
        pltpu.VMEM((SJ2, K_SEG), jnp.int32),
        pltpu.VMEM((SJ2, K_SEG), jnp.int32),
    ] + [pltpu.VMEM((K_SEG, FH), jnp.float32) for _ in range(3)] \
      + [pltpu.SemaphoreType.DMA for _ in range(6)]

    def body(hflat_hbm, src_hbm, dst_hbm, out_hbm,
             acc_s, sidx_blk, didx_blk, rv0, rv1, rv2,
             sg0, sg1, sg2, ss0, ss1, ss2):
        c = lax.axis_index("c")
        s = lax.axis_index("s")
        r0 = s * RPT
        rvs = (rv0, rv1, rv2)
        sgs = (sg0, sg1, sg2)
        sss = (ss0, ss1, ss2)
        off = jnp.zeros((16,), jnp.int32) + c * NPAD

        _fill(rv0, 0.0, K_SEG, FH)
        for zb in range(RPT // K_SEG):
            pltpu.sync_copy(rv0, acc_s.at[pl.ds(r0 + zb * K_SEG, K_SEG)])
        plsc.subcore_barrier()

        def sblk(u, carry):
            row0 = s * CPT + u * SJ2
            pltpu.sync_copy(src_hbm.at[pl.ds(row0, SJ2)], sidx_blk)
            pltpu.sync_copy(dst_hbm.at[pl.ds(row0, SJ2)], didx_blk)

            def offrow(r, carry2):
                for cc in range(K_SEG // 16):
                    sidx_blk[r, pl.ds(cc * 16, 16)] = (
                        sidx_blk[r, pl.ds(cc * 16, 16)] + off)
                return carry2

            lax.fori_loop(0, SJ2, offrow, 0)

            def tri(i, carry2):
                gs = []
                for b in range(3):
                    gs.append(pltpu.async_copy(
                        hflat_hbm.at[sidx_blk.at[3 * i + b]], rvs[b],
                        sgs[b]))
                scs = []
                for b in range(3):
                    gs[b].wait()
                    scs.append(pltpu.async_copy(
                        rvs[b], acc_s.at[didx_blk.at[3 * i + b]],
                        sss[b], add=True))
                for sp in scs:
                    sp.wait()
                return carry2

            lax.fori_loop(0, SJ2 // 3, tri, 0)
            return carry

        lax.fori_loop(0, CPT // SJ2, sblk, 0)
        plsc.subcore_barrier()

        pltpu.sync_copy(acc_s.at[pl.ds(r0, RPT)],
                        out_hbm.at[c, pl.ds(r0, RPT)])

    return pl.kernel(
        body,
        out_type=jax.ShapeDtypeStruct((NC, NPAD, FH), jnp.float32),
        mesh=mesh, scratch_types=scratch,
        compiler_params=pltpu.CompilerParams(use_tc_tiling_on_sc=False))


_segsum4 = _make_segsum_hbm()


RB = 512
NB = NPAD // RB


def _combine1_body(agg_ref, cnt_ref, x_ref, wl_ref, b_ref, wr_ref, out_ref):
    cnt = jnp.maximum(cnt_ref[...], 1.0)
    mean = jnp.concatenate([agg_ref[0], agg_ref[1]], axis=-1) / cnt
    h = (jnp.dot(mean, wl_ref[...], preferred_element_type=jnp.float32)
         + b_ref[...]
         + jnp.dot(x_ref[...], wr_ref[...],
                   preferred_element_type=jnp.float32))
    h = jnp.maximum(h, 0.0)
    for q in range(2):
        out_ref[q] = h[:, q * FH:(q + 1) * FH]


def _combine1(agg1, cnt2d, x_pad, W1_l, b1, W1_r):
    return pl.pallas_call(
        _combine1_body,
        grid=(NB,),
        in_specs=[
            pl.BlockSpec((2, RB, F), lambda i: (0, i, 0)),
            pl.BlockSpec((RB, 1), lambda i: (i, 0)),
            pl.BlockSpec((RB, IN_CH), lambda i: (i, 0)),
            pl.BlockSpec((IN_CH, HIDDEN), lambda i: (0, 0)),
            pl.BlockSpec((1, HIDDEN), lambda i: (0, 0)),
            pl.BlockSpec((IN_CH, HIDDEN), lambda i: (0, 0)),
        ],
        out_specs=pl.BlockSpec((2, RB, FH), lambda i: (0, i, 0)),
        out_shape=jax.ShapeDtypeStruct((2, NPAD, FH), jnp.float32),
    )(agg1, cnt2d, x_pad, W1_l, b1.reshape(1, HIDDEN), W1_r)


def _combine2_body(agg_ref, cnt_ref, h_ref, wl_ref, b_ref, wr_ref, out_ref):
    cnt = jnp.maximum(cnt_ref[...], 1.0)
    mean = jnp.concatenate([agg_ref[0], agg_ref[1]], axis=-1) / cnt
    h = jnp.concatenate([h_ref[0], h_ref[1]], axis=-1)
    out_ref[...] = (
        jnp.dot(mean, wl_ref[...], preferred_element_type=jnp.float32)
        + b_ref[...]
        + jnp.dot(h, wr_ref[...], preferred_element_type=jnp.float32))


def _combine2(agg2, cnt2d, hT2, W2_l, b2, W2_r):
    return pl.pallas_call(
        _combine2_body,
        grid=(NB,),
        in_specs=[
            pl.BlockSpec((2, RB, FH), lambda i: (0, i, 0)),
            pl.BlockSpec((RB, 1), lambda i: (i, 0)),
            pl.BlockSpec((2, RB, FH), lambda i: (0, i, 0)),
            pl.BlockSpec((HIDDEN, HIDDEN), lambda i: (0, 0)),
            pl.BlockSpec((1, HIDDEN), lambda i: (0, 0)),
            pl.BlockSpec((HIDDEN, HIDDEN), lambda i: (0, 0)),
        ],
        out_specs=pl.BlockSpec((RB, HIDDEN), lambda i: (i, 0)),
        out_shape=jax.ShapeDtypeStruct((NPAD, HIDDEN), jnp.float32),
    )(agg2, cnt2d, hT2, W2_l, b2.reshape(1, HIDDEN), W2_r)


def _make_decode():
    mesh = plsc.VectorSubcoreMesh(core_axis_name="c", subcore_axis_name="s")
    scratch = (
        [pltpu.VMEM((DCH, K_DEC), jnp.int32) for _ in range(2)]
        + [pltpu.VMEM((K_DEC, HIDDEN // 2), jnp.int32) for _ in range(8)]
        + [pltpu.VMEM((PPT,), jnp.float32)]
        + [pltpu.SemaphoreType.DMA for _ in range(8)]
    )

    def body(z_hbm, es_hbm, ed_hbm, out_hbm, sidx_all, didx_all,
             zs0, zd0, zs1, zd1, zs2, zd2, zs3, zd3, outv,
             ga0, gb0, ga1, gb1, ga2, gb2, ga3, gb3):
        c = lax.axis_index("c")
        s = lax.axis_index("s")
        w = c * NS + s
        lanes = lax.iota(jnp.int32, 16)
        z16 = jnp.zeros((16,), jnp.float32)
        zss = (zs0, zs1, zs2, zs3)
        zds = (zd0, zd1, zd2, zd3)
        gas = (ga0, ga1, ga2, ga3)
        gbs = (gb0, gb1, gb2, gb3)

        pltpu.sync_copy(es_hbm.at[pl.ds(w * DCH, DCH)], sidx_all)
        pltpu.sync_copy(ed_hbm.at[pl.ds(w * DCH, DCH)], didx_all)

        def compute(zs, zd, j):
            def pair_loop(g, carry):
                vec = z16
                for i in range(16):
                    idx = g * 16 + i
                    himask = jnp.full((16,), -65536, jnp.int32)
                    terms = []
                    for t in range(HIDDEN // 32):
                        wa = zs[idx, pl.ds(t * 16, 16)]
                        wb = zd[idx, pl.ds(t * 16, 16)]
                        la = plsc.bitcast(wa << 16, jnp.float32)
                        lb = plsc.bitcast(wb << 16, jnp.float32)
                        ha = plsc.bitcast(wa & himask, jnp.float32)
                        hb = plsc.bitcast(wb & himask, jnp.float32)
                        terms.append(la * lb + ha * hb)
                    while len(terms) > 1:
                        terms = [terms[k] + terms[k + 1]
                                 for k in range(0, len(terms) - 1, 2)] + (
                                     [terms[-1]] if len(terms) % 2 else [])
                    vec = jnp.where(lanes == i, jnp.sum(terms[0]), vec)
                outv[pl.ds(j * K_DEC + g * 16, 16)] = vec
                return carry

            lax.fori_loop(0, K_DEC // 16, pair_loop, 0)

        def it(i, carry):
            cps = []
            for b in range(4):
                j = 4 * i + b
                cps.append((
                    pltpu.async_copy(z_hbm.at[sidx_all.at[j]], zss[b],
                                     gas[b]),
                    pltpu.async_copy(z_hbm.at[didx_all.at[j]], zds[b],
                                     gbs[b]),
                ))
            for b in range(4):
                cps[b][0].wait()
                cps[b][1].wait()
                compute(zss[b], zds[b], 4 * i + b)
            return carry

        lax.fori_loop(0, DCH // 4, it, 0)

        pltpu.sync_copy(outv, out_hbm.at[pl.ds(w * PPT, PPT)])

    return pl.kernel(body,
                     out_type=jax.ShapeDtypeStruct((PPAD,), jnp.float32),
                     mesh=mesh, scratch_types=scratch,
                     compiler_params=pltpu.CompilerParams(
                         use_tc_tiling_on_sc=False,
                         needs_layout_passes=False))


_decode = _make_decode()


def kernel(x, edge_index, edges, W1_l, b1, W1_r, W2_l, b2, W2_r):
    pad_e = EPAD - E
    pad_src = (jnp.arange(pad_e, dtype=jnp.int32) * 97) % N
    pad_dst = N + (jnp.arange(pad_e, dtype=jnp.int32) % (NPAD - N))
    src2 = jnp.concatenate([edge_index[0], pad_src]).reshape(
        EPAD // K_SEG, K_SEG)
    dst2 = jnp.concatenate([edge_index[1], pad_dst]).reshape(
        EPAD // K_SEG, K_SEG)
    x_pad = jnp.pad(x, ((0, NPAD - N), (0, 0)))
    xT2 = x_pad.reshape(NPAD, 2, F).transpose(1, 0, 2)

    agg1, cnt = _segsum2(xT2, src2, dst2)
    cnt2d = cnt.reshape(NPAD, 1)
    hT = _combine1(agg1, cnt2d, x_pad, W1_l, b1, W1_r)
    agg2 = _segsum4(hT.reshape(NC * NPAD, FH), src2, dst2)
    z = _combine2(agg2, cnt2d, hT, W2_l, b2, W2_r)

    pad_p = PPAD - P
    pad_idx = (jnp.arange(pad_p, dtype=jnp.int32) * 89) % N
    es2 = jnp.concatenate([edges[:, 0], pad_idx]).reshape(
        PPAD // K_DEC, K_DEC)
    ed2 = jnp.concatenate([edges[:, 1], pad_idx]).reshape(
        PPAD // K_DEC, K_DEC)
    zi = lax.bitcast_convert_type(
        z.astype(jnp.bfloat16).reshape(NPAD, HIDDEN // 2, 2), jnp.int32)
    out = _decode(zi, es2, ed2)
    return out[:P]


# speedup vs baseline: 1.0346x; 1.0104x over previous
import functools

import jax
import jax.numpy as jnp
from jax import lax
from jax.experimental import pallas as pl
from jax.experimental.pallas import tpu as pltpu
from jax.experimental.pallas import tpu_sc as plsc

N = 10000
E = 320000
P = 100000
IN_CH = 128
HIDDEN = 256

NC = 2
NS = 16
NPAD = 10240
RPT = NPAD // NS
F = 64

K_SEG = 80
CPT = 252
EPT = K_SEG * CPT
EPAD = EPT * NS
SJ = 36
SB = CPT // SJ

K_DEC = 96
DCH = 36
PPT = K_DEC * DCH
PPAD = PPT * NC * NS


def _fill(ref, val, rows, cols):
    v = jnp.full((16,), val, jnp.float32)

    def row(r, carry):
        def col(k, carry2):
            ref[r, pl.ds(k * 16, 16)] = v
            return carry2
        return lax.fori_loop(0, cols // 16, col, carry)

    lax.fori_loop(0, rows, row, 0)


def _fill_1d(ref, val, n):
    v = jnp.full((16,), val, jnp.float32)

    def it(k, carry):
        ref[pl.ds(k * 16, 16)] = v
        return carry

    lax.fori_loop(0, n // 16, it, 0)


def _make_segsum(nq, with_cnt):
    qpc = nq // NC
    mesh = plsc.VectorSubcoreMesh(core_axis_name="c", subcore_axis_name="s")

    out_type = [jax.ShapeDtypeStruct((nq, NPAD, F), jnp.float32)]
    if with_cnt:
        out_type.append(jax.ShapeDtypeStruct((NPAD,), jnp.float32))

    scratch = [
        pltpu.VMEM_SHARED((NPAD, F), jnp.float32),
        pltpu.VMEM_SHARED((NPAD, F), jnp.float32),
        pltpu.VMEM((SJ, K_SEG), jnp.int32),
        pltpu.VMEM((SJ, K_SEG), jnp.int32),
    ] + [pltpu.VMEM((K_SEG, F), jnp.float32) for _ in range(4)] \
      + [pltpu.SemaphoreType.DMA for _ in range(8)]
    if with_cnt:
        scratch += [
            pltpu.VMEM_SHARED((NPAD,), jnp.float32),
            pltpu.VMEM((K_SEG,), jnp.float32),
        ] + [pltpu.SemaphoreType.DMA for _ in range(4)]

    def body(tab_hbm, src_hbm, dst_hbm, out_hbm, *rest):
        if with_cnt:
            (cnt_hbm, tab_s, acc_s, sidx_blk, didx_blk,
             rv0, rv1, rv2, rv3,
             sg0, sg1, sg2, sg3, ss0, ss1, ss2, ss3,
             cnt_s, ones_v, sc0, sc1, sc2, sc3) = rest
        else:
            (tab_s, acc_s, sidx_blk, didx_blk,
             rv0, rv1, rv2, rv3,
             sg0, sg1, sg2, sg3, ss0, ss1, ss2, ss3) = rest
        c = lax.axis_index("c")
        s = lax.axis_index("s")
        r0 = s * RPT
        rvs = (rv0, rv1, rv2, rv3)
        sgs = (sg0, sg1, sg2, sg3)
        sss = (ss0, ss1, ss2, ss3)

        for qi in range(qpc):
            q = c * qpc + qi
            _fill(rv0, 0.0, K_SEG, F)
            pltpu.sync_copy(tab_hbm.at[q, pl.ds(r0, RPT)],
                            tab_s.at[pl.ds(r0, RPT)])
            for zb in range(RPT // K_SEG):
                pltpu.sync_copy(rv0,
                                acc_s.at[pl.ds(r0 + zb * K_SEG, K_SEG)])
            if with_cnt and qi == 0:
                _fill_1d(ones_v, 0.0, K_SEG)

                @pl.when(c == 0)
                def _():
                    for zb in range(RPT // K_SEG):
                        pltpu.sync_copy(
                            ones_v, cnt_s.at[pl.ds(r0 + zb * K_SEG, K_SEG)])
                _fill_1d(ones_v, 1.0, K_SEG)
            plsc.subcore_barrier()

            def sblk(u, carry):
                row0 = s * CPT + u * SJ
                pltpu.sync_copy(src_hbm.at[pl.ds(row0, SJ)], sidx_blk)
                pltpu.sync_copy(dst_hbm.at[pl.ds(row0, SJ)], didx_blk)

                def quad(i, carry2):
                    gs = []
                    for b in range(4):
                        gs.append(pltpu.async_copy(
                            tab_s.at[sidx_blk.at[4 * i + b]], rvs[b],
                            sgs[b]))
                    scs = []
                    for b in range(4):
                        gs[b].wait()
                        scs.append(pltpu.async_copy(
                            rvs[b], acc_s.at[didx_blk.at[4 * i + b]],
                            sss[b], add=True))
                    if with_cnt and qi == 0:
                        @pl.when(c == 0)
                        def _():
                            ccs = [pltpu.async_copy(
                                ones_v, cnt_s.at[didx_blk.at[4 * i + b]],
                                (sc0, sc1, sc2, sc3)[b], add=True)
                                for b in range(4)]
                            for cc in ccs:
                                cc.wait()
                    for sp in scs:
                        sp.wait()
                    return carry2

                lax.fori_loop(0, SJ // 4, quad, 0)
                return carry

            lax.fori_loop(0, SB, sblk, 0)
            plsc.subcore_barrier()

            pltpu.sync_copy(acc_s.at[pl.ds(r0, RPT)],
                            out_hbm.at[q, pl.ds(r0, RPT)])
            if with_cnt and qi == 0:
                @pl.when(c == 0)
                def _():
                    pltpu.sync_copy(cnt_s.at[pl.ds(r0, RPT)],
                                    cnt_hbm.at[pl.ds(r0, RPT)])

    return pl.kernel(body, out_type=tuple(out_type), mesh=mesh,
                     scratch_types=scratch,
                     compiler_params=pltpu.CompilerParams(
                         use_tc_tiling_on_sc=False))


_segsum2 = _make_segsum(2, True)


FH = 128
SJ2 = 28


def _make_segsum_hbm():
    mesh = plsc.VectorSubcoreMesh(core_axis_name="c", subcore_axis_name="s")
    scratch = [
        pltpu.VMEM_SHARED((NPAD, FH), jnp.float32),
        pltpu.VMEM((SJ2, K_SEG), jnp.int32),
        pltpu.VMEM((SJ2, K_SEG), jnp.int32),
    ] + [pltpu.VMEM((K_SEG, FH), jnp.float32) for _ in range(4)] \
      + [pltpu.SemaphoreType.DMA for _ in range(8)]

    def body(hflat_hbm, src_hbm, dst_hbm, out_hbm,
             acc_s, sidx_blk, didx_blk, rv0, rv1, rv2, rv3,
             sg0, sg1, sg2, sg3, ss0, ss1, ss2, ss3):
        c = lax.axis_index("c")
        s = lax.axis_index("s")
        r0 = s * RPT
        rvs = (rv0, rv1, rv2, rv3)
        sgs = (sg0, sg1, sg2, sg3)
        sss = (ss0, ss1, ss2, ss3)
        off = jnp.zeros((16,), jnp.int32) + c * NPAD

        _fill(rv0, 0.0, K_SEG, FH)
        for zb in range(RPT // K_SEG):
            pltpu.sync_copy(rv0, acc_s.at[pl.ds(r0 + zb * K_SEG, K_SEG)])
        plsc.subcore_barrier()

        def sblk(u, carry):
            row0 = s * CPT + u * SJ2
            pltpu.sync_copy(src_hbm.at[pl.ds(row0, SJ2)], sidx_blk)
            pltpu.sync_copy(dst_hbm.at[pl.ds(row0, SJ2)], didx_blk)

            def offrow(r, carry2):
                for cc in range(K_SEG // 16):
                    sidx_blk[r, pl.ds(cc * 16, 16)] = (
                        sidx_blk[r, pl.ds(cc * 16, 16)] + off)
                return carry2

            lax.fori_loop(0, SJ2, offrow, 0)

            def quad(i, carry2):
                gs = []
                for b in range(4):
                    gs.append(pltpu.async_copy(
                        hflat_hbm.at[sidx_blk.at[4 * i + b]], rvs[b],
                        sgs[b]))
                scs = []
                for b in range(4):
                    gs[b].wait()
                    scs.append(pltpu.async_copy(
                        rvs[b], acc_s.at[didx_blk.at[4 * i + b]],
                        sss[b], add=True))
                for sp in scs:
                    sp.wait()
                return carry2

            lax.fori_loop(0, SJ2 // 4, quad, 0)
            return carry

        lax.fori_loop(0, CPT // SJ2, sblk, 0)
        plsc.subcore_barrier()

        pltpu.sync_copy(acc_s.at[pl.ds(r0, RPT)],
                        out_hbm.at[c, pl.ds(r0, RPT)])

    return pl.kernel(
        body,
        out_type=jax.ShapeDtypeStruct((NC, NPAD, FH), jnp.float32),
        mesh=mesh, scratch_types=scratch,
        compiler_params=pltpu.CompilerParams(use_tc_tiling_on_sc=False))


_segsum4 = _make_segsum_hbm()


RB = 512
NB = NPAD // RB


def _combine1_body(agg_ref, cnt_ref, x_ref, wl_ref, b_ref, wr_ref, out_ref):
    cnt = jnp.maximum(cnt_ref[...], 1.0)
    mean = jnp.concatenate([agg_ref[0], agg_ref[1]], axis=-1) / cnt
    h = (jnp.dot(mean, wl_ref[...], preferred_element_type=jnp.float32)
         + b_ref[...]
         + jnp.dot(x_ref[...], wr_ref[...],
                   preferred_element_type=jnp.float32))
    h = jnp.maximum(h, 0.0)
    for q in range(2):
        out_ref[q] = h[:, q * FH:(q + 1) * FH]


def _combine1(agg1, cnt2d, x_pad, W1_l, b1, W1_r):
    return pl.pallas_call(
        _combine1_body,
        grid=(NB,),
        in_specs=[
            pl.BlockSpec((2, RB, F), lambda i: (0, i, 0)),
            pl.BlockSpec((RB, 1), lambda i: (i, 0)),
            pl.BlockSpec((RB, IN_CH), lambda i: (i, 0)),
            pl.BlockSpec((IN_CH, HIDDEN), lambda i: (0, 0)),
            pl.BlockSpec((1, HIDDEN), lambda i: (0, 0)),
            pl.BlockSpec((IN_CH, HIDDEN), lambda i: (0, 0)),
        ],
        out_specs=pl.BlockSpec((2, RB, FH), lambda i: (0, i, 0)),
        out_shape=jax.ShapeDtypeStruct((2, NPAD, FH), jnp.float32),
    )(agg1, cnt2d, x_pad, W1_l, b1.reshape(1, HIDDEN), W1_r)


def _combine2_body(agg_ref, cnt_ref, h_ref, wl_ref, b_ref, wr_ref, out_ref):
    cnt = jnp.maximum(cnt_ref[...], 1.0)
    mean = jnp.concatenate([agg_ref[0], agg_ref[1]], axis=-1) / cnt
    h = jnp.concatenate([h_ref[0], h_ref[1]], axis=-1)
    out_ref[...] = (
        jnp.dot(mean, wl_ref[...], preferred_element_type=jnp.float32)
        + b_ref[...]
        + jnp.dot(h, wr_ref[...], preferred_element_type=jnp.float32))


def _combine2(agg2, cnt2d, hT2, W2_l, b2, W2_r):
    return pl.pallas_call(
        _combine2_body,
        grid=(NB,),
        in_specs=[
            pl.BlockSpec((2, RB, FH), lambda i: (0, i, 0)),
            pl.BlockSpec((RB, 1), lambda i: (i, 0)),
            pl.BlockSpec((2, RB, FH), lambda i: (0, i, 0)),
            pl.BlockSpec((HIDDEN, HIDDEN), lambda i: (0, 0)),
            pl.BlockSpec((1, HIDDEN), lambda i: (0, 0)),
            pl.BlockSpec((HIDDEN, HIDDEN), lambda i: (0, 0)),
        ],
        out_specs=pl.BlockSpec((RB, HIDDEN), lambda i: (i, 0)),
        out_shape=jax.ShapeDtypeStruct((NPAD, HIDDEN), jnp.float32),
    )(agg2, cnt2d, hT2, W2_l, b2.reshape(1, HIDDEN), W2_r)


def _make_decode():
    mesh = plsc.VectorSubcoreMesh(core_axis_name="c", subcore_axis_name="s")
    scratch = (
        [pltpu.VMEM((DCH, K_DEC), jnp.int32) for _ in range(2)]
        + [pltpu.VMEM((K_DEC, HIDDEN // 2), jnp.int32) for _ in range(8)]
        + [pltpu.VMEM((PPT,), jnp.float32)]
        + [pltpu.SemaphoreType.DMA for _ in range(8)]
    )

    def body(z_hbm, es_hbm, ed_hbm, out_hbm, sidx_all, didx_all,
             zs0, zd0, zs1, zd1, zs2, zd2, zs3, zd3, outv,
             ga0, gb0, ga1, gb1, ga2, gb2, ga3, gb3):
        c = lax.axis_index("c")
        s = lax.axis_index("s")
        w = c * NS + s
        lanes = lax.iota(jnp.int32, 16)
        z16 = jnp.zeros((16,), jnp.float32)
        zss = (zs0, zs1, zs2, zs3)
        zds = (zd0, zd1, zd2, zd3)
        gas = (ga0, ga1, ga2, ga3)
        gbs = (gb0, gb1, gb2, gb3)

        pltpu.sync_copy(es_hbm.at[pl.ds(w * DCH, DCH)], sidx_all)
        pltpu.sync_copy(ed_hbm.at[pl.ds(w * DCH, DCH)], didx_all)

        def compute(zs, zd, j):
            def pair_loop(g, carry):
                vec = z16
                for i in range(16):
                    idx = g * 16 + i
                    himask = jnp.full((16,), -65536, jnp.int32)
                    terms = []
                    for t in range(HIDDEN // 32):
                        wa = zs[idx, pl.ds(t * 16, 16)]
                        wb = zd[idx, pl.ds(t * 16, 16)]
                        la = plsc.bitcast(wa << 16, jnp.float32)
                        lb = plsc.bitcast(wb << 16, jnp.float32)
                        ha = plsc.bitcast(wa & himask, jnp.float32)
                        hb = plsc.bitcast(wb & himask, jnp.float32)
                        terms.append(la * lb + ha * hb)
                    while len(terms) > 1:
                        terms = [terms[k] + terms[k + 1]
                                 for k in range(0, len(terms) - 1, 2)] + (
                                     [terms[-1]] if len(terms) % 2 else [])
                    vec = jnp.where(lanes == i, jnp.sum(terms[0]), vec)
                outv[pl.ds(j * K_DEC + g * 16, 16)] = vec
                return carry

            lax.fori_loop(0, K_DEC // 16, pair_loop, 0)

        def it(i, carry):
            cps = []
            for b in range(4):
                j = 4 * i + b
                cps.append((
                    pltpu.async_copy(z_hbm.at[sidx_all.at[j]], zss[b],
                                     gas[b]),
                    pltpu.async_copy(z_hbm.at[didx_all.at[j]], zds[b],
                                     gbs[b]),
                ))
            for b in range(4):
                cps[b][0].wait()
                cps[b][1].wait()
                compute(zss[b], zds[b], 4 * i + b)
            return carry

        lax.fori_loop(0, DCH // 4, it, 0)

        pltpu.sync_copy(outv, out_hbm.at[pl.ds(w * PPT, PPT)])

    return pl.kernel(body,
                     out_type=jax.ShapeDtypeStruct((PPAD,), jnp.float32),
                     mesh=mesh, scratch_types=scratch,
                     compiler_params=pltpu.CompilerParams(
                         use_tc_tiling_on_sc=False,
                         needs_layout_passes=False))


_decode = _make_decode()


def kernel(x, edge_index, edges, W1_l, b1, W1_r, W2_l, b2, W2_r):
    pad_e = EPAD - E
    pad_src = (jnp.arange(pad_e, dtype=jnp.int32) * 97) % N
    pad_dst = N + (jnp.arange(pad_e, dtype=jnp.int32) % (NPAD - N))
    src2 = jnp.concatenate([edge_index[0], pad_src]).reshape(
        EPAD // K_SEG, K_SEG)
    dst2 = jnp.concatenate([edge_index[1], pad_dst]).reshape(
        EPAD // K_SEG, K_SEG)
    x_pad = jnp.pad(x, ((0, NPAD - N), (0, 0)))
    xT2 = x_pad.reshape(NPAD, 2, F).transpose(1, 0, 2)

    agg1, cnt = _segsum2(xT2, src2, dst2)
    cnt2d = cnt.reshape(NPAD, 1)
    hT = _combine1(agg1, cnt2d, x_pad, W1_l, b1, W1_r)
    agg2 = _segsum4(hT.reshape(NC * NPAD, FH), src2, dst2)
    z = _combine2(agg2, cnt2d, hT, W2_l, b2, W2_r)

    pad_p = PPAD - P
    pad_idx = (jnp.arange(pad_p, dtype=jnp.int32) * 89) % N
    es2 = jnp.concatenate([edges[:, 0], pad_idx]).reshape(
        PPAD // K_DEC, K_DEC)
    ed2 = jnp.concatenate([edges[:, 1], pad_idx]).reshape(
        PPAD // K_DEC, K_DEC)
    zi = lax.bitcast_convert_type(
        z.astype(jnp.bfloat16).reshape(NPAD, HIDDEN // 2, 2), jnp.int32)
    out = _decode(zi, es2, ed2)
    return out[:P]
